# baseline (speedup 1.0000x reference)
import math

import jax
import jax.numpy as jnp
from jax import lax
from jax.experimental import pallas as pl
from jax.experimental.pallas import tpu as pltpu
from jax.experimental.pallas import tpu_sc as plsc

N = 10000
E = 160000
D = 512
C_CURV = 1.0
NCHUNK = 4
CHUNK = D // NCHUNK
BATCH = 128
HB = BATCH // 2
NC = 2
NS = 16
NB_TILE = 80
E_PAD = NB_TILE * BATCH * NS
ROWS_TILE = 640
N_PAD = ROWS_TILE * NS

BN = 1000


def _linear_body(x_ref, w_ref, b_ref, ls_ref, *hc_refs):
    h = lax.dot_general(x_ref[...], w_ref[...], (((1,), (1,)), ((), ())),
                        preferred_element_type=jnp.float32)
    h = h + b_ref[...]
    scale0 = jnp.exp(ls_ref[0, 0])
    time = jax.nn.sigmoid(h[:, 0:1]) * scale0 + (math.sqrt(C_CURV) + 0.5)
    x_narrow = h[:, 1:]
    sq = jnp.clip(jnp.sum(x_narrow * x_narrow, axis=-1, keepdims=True),
                  1e-8, None)
    sc = (time * time - C_CURV) / sq
    row = jnp.concatenate(
        [time, x_narrow * jnp.sqrt(jnp.clip(sc, 1e-8, None))], axis=-1)
    for cc in range(NCHUNK):
        hc_refs[cc][...] = row[:, cc * CHUNK:(cc + 1) * CHUNK]


def _lorentz_linear_chunked(x, W, b2, ls2):
    return pl.pallas_call(
        _linear_body,
        grid=(N // BN,),
        in_specs=[
            pl.BlockSpec((BN, D), lambda i: (i, 0)),
            pl.BlockSpec((D, D), lambda i: (0, 0)),
            pl.BlockSpec((1, D), lambda i: (0, 0)),
            pl.BlockSpec(memory_space=pltpu.SMEM),
        ],
        out_specs=[pl.BlockSpec((BN, CHUNK), lambda i: (i, 0))
                   for _ in range(NCHUNK)],
        out_shape=[jax.ShapeDtypeStruct((N, CHUNK), jnp.float32)
                   for _ in range(NCHUNK)],
    )(x, W, b2, ls2)


def _norm_body(*refs):
    s_refs = refs[:NCHUNK]
    out_ref = refs[NCHUNK]
    total_sq = jnp.zeros((BN, 1), jnp.float32)
    for cc in range(NCHUNK):
        s = s_refs[cc][...]
        total_sq = total_sq + jnp.sum(s * s, axis=-1, keepdims=True)
    t0 = s_refs[0][:, 0:1]
    inner = total_sq - 2.0 * t0 * t0
    denom = jnp.sqrt(jnp.clip(jnp.abs(inner), 1e-8, None)) / math.sqrt(C_CURV)
    inv = 1.0 / denom
    for cc in range(NCHUNK):
        out_ref[:, cc * CHUNK:(cc + 1) * CHUNK] = s_refs[cc][...] * inv


def _lorentz_normalize(sup):
    return pl.pallas_call(
        _norm_body,
        grid=(N // BN,),
        in_specs=[pl.BlockSpec((BN, CHUNK), lambda i: (i, 0))
                  for _ in range(NCHUNK)],
        out_specs=pl.BlockSpec((BN, D), lambda i: (i, 0)),
        out_shape=jax.ShapeDtypeStruct((N, D), jnp.float32),
    )(*sup)


def _aggregate_body(*refs):
    h_refs = refs[0:NCHUNK]
    src_hbm, dst_hbm, w_hbm, z_hbm = refs[NCHUNK:NCHUNK + 4]
    out_refs = refs[NCHUNK + 4:2 * NCHUNK + 4]
    k = 2 * NCHUNK + 4
    acc, idx_l = refs[k:k + 2]
    rows = refs[k + 2:k + 4]
    dstb = refs[k + 4:k + 6]
    wb = refs[k + 6:k + 8]
    gsem = (refs[k + 8:k + 10], refs[k + 10:k + 12])
    esem = refs[k + 12:k + 14]
    ssem = refs[k + 14:k + 16]

    core = lax.axis_index("c")
    sub = lax.axis_index("s")

    pltpu.sync_copy(src_hbm.at[sub], idx_l)

    def g_desc(hk, b, p, half):
        return pltpu.make_async_copy(
            hk.at[idx_l.at[b, pl.ds(half * HB, HB)]],
            rows[p].at[pl.ds(half * HB, HB)], gsem[p][half])

    def e_descs(b, p):
        return (pltpu.make_async_copy(dst_hbm.at[sub, b], dstb[p], esem[p]),
                pltpu.make_async_copy(w_hbm.at[sub, b], wb[p], esem[p]))

    def s_desc(b, p):
        return pltpu.make_async_copy(rows[p], acc.at[dstb[p].at[0]], ssem[p])

    for chunk in range(NCHUNK):
        @pl.when(core == chunk // (NCHUNK // NC))
        def _pass():
            hk = h_refs[chunk]
            ok = out_refs[chunk]
            pltpu.sync_copy(z_hbm, acc.at[pl.ds(sub * ROWS_TILE, ROWS_TILE)])
            plsc.subcore_barrier()

            for d in e_descs(0, 0):
                d.start()
            g_desc(hk, 0, 0, 0).start()
            g_desc(hk, 0, 0, 1).start()

            def pair_step(g, _):
                for p in range(2):
                    b = g * 2 + p
                    o = 1 - p

                    @pl.when(b >= 1)
                    def _wait_prev_scatter():
                        s_desc(b - 1, o).wait()

                    @pl.when(b + 1 < NB_TILE)
                    def _issue_next():
                        for d in e_descs(b + 1, o):
                            d.start()
                        g_desc(hk, b + 1, o, 0).start()
                        g_desc(hk, b + 1, o, 1).start()

                    g_desc(hk, b, p, 0).wait()
                    g_desc(hk, b, p, 1).wait()
                    for d in e_descs(b, p):
                        d.wait()

                    def mul_step(gg, _):
                        wvec = wb[p][0, pl.ds(gg * 16, 16)]
                        for l in range(16):
                            wj = wvec[l]
                            j = gg * 16 + l
                            for i in range(CHUNK // 16):
                                sl = pl.ds(i * 16, 16)
                                rows[p][j, sl] = rows[p][j, sl] * wj
                        return ()
                    lax.fori_loop(0, BATCH // 16, mul_step, ())

                    pltpu.async_copy(rows[p], acc.at[dstb[p].at[0]],
                                     ssem[p], add=True)
                return ()

            lax.fori_loop(0, NB_TILE // 2, pair_step, ())
            s_desc(NB_TILE - 1, 1).wait()
            plsc.subcore_barrier()
            pltpu.sync_copy(acc.at[pl.ds(sub * ROWS_TILE, ROWS_TILE)],
                            ok.at[pl.ds(sub * ROWS_TILE, ROWS_TILE)])
            plsc.subcore_barrier()


def _aggregate(hs, src3, dst4, w4, z):
    mesh = plsc.VectorSubcoreMesh(core_axis_name="c", subcore_axis_name="s")
    kfn = pl.kernel(
        _aggregate_body,
        out_type=[jax.ShapeDtypeStruct((N_PAD, CHUNK), jnp.float32)
                  for _ in range(NCHUNK)],
        mesh=mesh,
        scratch_types=(
            [
                pltpu.VMEM_SHARED((N_PAD, CHUNK), jnp.float32),
                pltpu.VMEM((NB_TILE, BATCH), jnp.int32),
                pltpu.VMEM((BATCH, CHUNK), jnp.float32),
                pltpu.VMEM((BATCH, CHUNK), jnp.float32),
                pltpu.VMEM((1, BATCH), jnp.int32),
                pltpu.VMEM((1, BATCH), jnp.int32),
                pltpu.VMEM((1, BATCH), jnp.float32),
                pltpu.VMEM((1, BATCH), jnp.float32),
            ]
            + [pltpu.SemaphoreType.DMA for _ in range(8)]
        ),
    )
    return kfn(*hs, src3, dst4, w4, z)


@jax.jit
def kernel(x, edge_index, edge_weight, W, b, log_scale):
    b2 = b.reshape(1, D)
    ls2 = log_scale.reshape(1, 1)
    hs = _lorentz_linear_chunked(x.astype(jnp.bfloat16), W.astype(jnp.bfloat16), b2, ls2)

    pad = E_PAD - E
    spread = jnp.arange(pad, dtype=jnp.int32) % N
    src = jnp.concatenate([edge_index[1].astype(jnp.int32), spread])
    dst = jnp.concatenate([edge_index[0].astype(jnp.int32), spread])
    w = jnp.concatenate([edge_weight, jnp.zeros((pad,), jnp.float32)])
    src3 = src.reshape(NS, NB_TILE, BATCH)
    dst4 = dst.reshape(NS, NB_TILE, 1, BATCH)
    w4 = w.reshape(NS, NB_TILE, 1, BATCH)
    z = jnp.zeros((ROWS_TILE, CHUNK), jnp.float32)

    sup = _aggregate(hs, src3, dst4, w4, z)
    sup = [s[:N] for s in sup]
    return _lorentz_normalize(sup)


# speedup vs baseline: 1.0381x; 1.0381x over previous
import math

import jax
import jax.numpy as jnp
from jax import lax
from jax.experimental import pallas as pl
from jax.experimental.pallas import tpu as pltpu
from jax.experimental.pallas import tpu_sc as plsc

N = 10000
E = 160000
D = 512
C_CURV = 1.0
NCHUNK = 4
CHUNK = D // NCHUNK
BATCH = 128
HB = BATCH // 2
NC = 2
NS = 16
NB_TILE = 80
E_PAD = NB_TILE * BATCH * NS
ROWS_TILE = 640
N_PAD = ROWS_TILE * NS

BN = 1000


def _linear_body(x_ref, w_ref, b_ref, ls_ref, *hc_refs):
    h = lax.dot_general(x_ref[...], w_ref[...], (((1,), (1,)), ((), ())),
                        preferred_element_type=jnp.float32)
    h = h + b_ref[...]
    scale0 = jnp.exp(ls_ref[0, 0])
    time = jax.nn.sigmoid(h[:, 0:1]) * scale0 + (math.sqrt(C_CURV) + 0.5)
    x_narrow = h[:, 1:]
    sq = jnp.clip(jnp.sum(x_narrow * x_narrow, axis=-1, keepdims=True),
                  1e-8, None)
    sc = (time * time - C_CURV) / sq
    row = jnp.concatenate(
        [time, x_narrow * jnp.sqrt(jnp.clip(sc, 1e-8, None))], axis=-1)
    for cc in range(NCHUNK):
        hc_refs[cc][...] = row[:, cc * CHUNK:(cc + 1) * CHUNK]


def _lorentz_linear_chunked(x, W, b2, ls2):
    return pl.pallas_call(
        _linear_body,
        grid=(N // BN,),
        in_specs=[
            pl.BlockSpec((BN, D), lambda i: (i, 0)),
            pl.BlockSpec((D, D), lambda i: (0, 0)),
            pl.BlockSpec((1, D), lambda i: (0, 0)),
            pl.BlockSpec(memory_space=pltpu.SMEM),
        ],
        out_specs=[pl.BlockSpec((BN, CHUNK), lambda i: (i, 0))
                   for _ in range(NCHUNK)],
        out_shape=[jax.ShapeDtypeStruct((N, CHUNK), jnp.float32)
                   for _ in range(NCHUNK)],
    )(x, W, b2, ls2)


def _norm_body(*refs):
    s_refs = refs[:NCHUNK]
    out_ref = refs[NCHUNK]
    total_sq = jnp.zeros((BN, 1), jnp.float32)
    for cc in range(NCHUNK):
        s = s_refs[cc][...]
        total_sq = total_sq + jnp.sum(s * s, axis=-1, keepdims=True)
    t0 = s_refs[0][:, 0:1]
    inner = total_sq - 2.0 * t0 * t0
    denom = jnp.sqrt(jnp.clip(jnp.abs(inner), 1e-8, None)) / math.sqrt(C_CURV)
    inv = 1.0 / denom
    for cc in range(NCHUNK):
        out_ref[:, cc * CHUNK:(cc + 1) * CHUNK] = s_refs[cc][...] * inv


def _lorentz_normalize(sup):
    return pl.pallas_call(
        _norm_body,
        grid=(N // BN,),
        in_specs=[pl.BlockSpec((BN, CHUNK), lambda i: (i, 0))
                  for _ in range(NCHUNK)],
        out_specs=pl.BlockSpec((BN, D), lambda i: (i, 0)),
        out_shape=jax.ShapeDtypeStruct((N, D), jnp.float32),
    )(*sup)


def _aggregate_body(*refs):
    h_refs = refs[0:NCHUNK]
    src_hbm, dst_hbm, w_hbm, z_hbm = refs[NCHUNK:NCHUNK + 4]
    out_refs = refs[NCHUNK + 4:2 * NCHUNK + 4]
    k = 2 * NCHUNK + 4
    acc, idx_l = refs[k:k + 2]
    rows = refs[k + 2:k + 4]
    dstb = refs[k + 4:k + 6]
    wb = refs[k + 6:k + 8]
    gsem = (refs[k + 8:k + 10], refs[k + 10:k + 12])
    esem = refs[k + 12:k + 14]
    ssem = refs[k + 14:k + 16]

    core = lax.axis_index("c")
    sub = lax.axis_index("s")

    pltpu.sync_copy(src_hbm.at[sub], idx_l)

    def g_desc(hk, b, p, half):
        return pltpu.make_async_copy(
            hk.at[idx_l.at[b, pl.ds(half * HB, HB)]],
            rows[p].at[pl.ds(half * HB, HB)], gsem[p][half])

    def e_descs(b, p):
        return (pltpu.make_async_copy(dst_hbm.at[sub, b], dstb[p], esem[p]),
                pltpu.make_async_copy(w_hbm.at[sub, b], wb[p], esem[p]))

    def s_desc(b, p):
        return pltpu.make_async_copy(rows[p], acc.at[dstb[p].at[0]], ssem[p])

    for chunk in range(NCHUNK):
        @pl.when(core == chunk // (NCHUNK // NC))
        def _pass():
            hk = h_refs[chunk]
            ok = out_refs[chunk]
            pltpu.sync_copy(z_hbm, acc.at[pl.ds(sub * ROWS_TILE, ROWS_TILE)])
            plsc.subcore_barrier()

            for d in e_descs(0, 0):
                d.start()
            g_desc(hk, 0, 0, 0).start()
            g_desc(hk, 0, 0, 1).start()

            def pair_step(g, _):
                for p in range(2):
                    b = g * 2 + p
                    o = 1 - p

                    @pl.when(b >= 1)
                    def _wait_prev_scatter():
                        s_desc(b - 1, o).wait()

                    @pl.when(b + 1 < NB_TILE)
                    def _issue_next():
                        for d in e_descs(b + 1, o):
                            d.start()
                        g_desc(hk, b + 1, o, 0).start()
                        g_desc(hk, b + 1, o, 1).start()

                    g_desc(hk, b, p, 0).wait()
                    g_desc(hk, b, p, 1).wait()
                    for d in e_descs(b, p):
                        d.wait()

                    def mul_step(gg, _):
                        wvec = wb[p][0, pl.ds(gg * 16, 16)]
                        for l in range(16):
                            wj = wvec[l]
                            j = gg * 16 + l
                            for i in range(CHUNK // 16):
                                sl = pl.ds(i * 16, 16)
                                rows[p][j, sl] = rows[p][j, sl] * wj
                        return ()
                    lax.fori_loop(0, BATCH // 16, mul_step, ())

                    pltpu.async_copy(rows[p], acc.at[dstb[p].at[0]],
                                     ssem[p], add=True)
                return ()

            lax.fori_loop(0, NB_TILE // 2, pair_step, ())
            s_desc(NB_TILE - 1, 1).wait()
            plsc.subcore_barrier()
            pltpu.sync_copy(acc.at[pl.ds(sub * ROWS_TILE, ROWS_TILE)],
                            ok.at[pl.ds(sub * ROWS_TILE, ROWS_TILE)])
            plsc.subcore_barrier()


def _aggregate(hs, src3, dst4, w4, z):
    mesh = plsc.VectorSubcoreMesh(core_axis_name="c", subcore_axis_name="s")
    kfn = pl.kernel(
        _aggregate_body,
        out_type=[jax.ShapeDtypeStruct((N_PAD, CHUNK), jnp.float32)
                  for _ in range(NCHUNK)],
        mesh=mesh,
        scratch_types=(
            [
                pltpu.VMEM_SHARED((N_PAD, CHUNK), jnp.float32),
                pltpu.VMEM((NB_TILE, BATCH), jnp.int32),
                pltpu.VMEM((BATCH, CHUNK), jnp.float32),
                pltpu.VMEM((BATCH, CHUNK), jnp.float32),
                pltpu.VMEM((1, BATCH), jnp.int32),
                pltpu.VMEM((1, BATCH), jnp.int32),
                pltpu.VMEM((1, BATCH), jnp.float32),
                pltpu.VMEM((1, BATCH), jnp.float32),
            ]
            + [pltpu.SemaphoreType.DMA for _ in range(8)]
        ),
    )
    return kfn(*hs, src3, dst4, w4, z)


@jax.jit
def kernel(x, edge_index, edge_weight, W, b, log_scale):
    b2 = b.reshape(1, D)
    ls2 = log_scale.reshape(1, 1)
    hs = _lorentz_linear_chunked(x, W, b2, ls2)

    pad = E_PAD - E
    spread = jnp.arange(pad, dtype=jnp.int32) % N
    src = jnp.concatenate([edge_index[1].astype(jnp.int32), spread])
    dst = jnp.concatenate([edge_index[0].astype(jnp.int32), spread])
    w = jnp.concatenate([edge_weight, jnp.zeros((pad,), jnp.float32)])
    src3 = src.reshape(NS, NB_TILE, BATCH)
    dst4 = dst.reshape(NS, NB_TILE, 1, BATCH)
    w4 = w.reshape(NS, NB_TILE, 1, BATCH)
    z = jnp.zeros((ROWS_TILE, CHUNK), jnp.float32)

    sup = _aggregate(hs, src3, dst4, w4, z)
    sup = [s[:N] for s in sup]
    return _lorentz_normalize(sup)
